# baseline (speedup 1.0000x reference)
import jax
import jax.numpy as jnp
from jax import lax
from jax.experimental import pallas as pl
from jax.experimental.pallas import tpu as pltpu
from jax.experimental.pallas import tpu_sc as plsc

N = 100000
D = 128
G = 512
RB = 2000
NSTEP = N // RB

NW = 32
P = 3136
NPAD = NW * P
CH = P // 16
BINS = G + 16


def _rowval_body(x_ref, g_ref, wt_ref, o_ref):
    xb = x_ref[...]
    u = g_ref[...] * wt_ref[...]
    su = jnp.sum(u)
    up = (u - su * (1.0 / D)) * (D ** 0.5)
    s1 = jnp.sum(xb, axis=1, keepdims=True)
    s2 = jnp.sum(xb * xb, axis=1, keepdims=True)
    sdc = jnp.sum(xb * up, axis=1, keepdims=True)
    denom = s2 - s1 * s1 * (1.0 / D) + (D * 1e-5)
    o_ref[...] = sdc * lax.rsqrt(denom)


_rowval_call = pl.pallas_call(
    _rowval_body,
    grid=(NSTEP,),
    in_specs=[
        pl.BlockSpec((RB, D), lambda i: (i, 0)),
        pl.BlockSpec((1, D), lambda i: (0, 0)),
        pl.BlockSpec((1, D), lambda i: (0, 0)),
    ],
    out_specs=pl.BlockSpec((RB, 1), lambda i: (i, 0)),
    out_shape=jax.ShapeDtypeStruct((NPAD, 1), jnp.float32),
)


def _sc_seg_body(rv_hbm, idx_hbm, sums_out, cnts_out, rv_v, idx_v, sums_v, cnts_v):
    wid = lax.axis_index("s") * 2 + lax.axis_index("c")
    base = wid * P
    pltpu.sync_copy(rv_hbm.at[pl.ds(base, P)], rv_v)
    pltpu.sync_copy(idx_hbm.at[pl.ds(base, P)], idx_v)

    z16 = jnp.zeros((16,), jnp.float32)
    one16 = jnp.ones((16,), jnp.float32)

    def zero_body(j, carry):
        sums_v[pl.ds(j * 16, 16)] = z16
        cnts_v[pl.ds(j * 16, 16)] = z16
        return carry

    lax.fori_loop(0, BINS // 16, zero_body, 0)

    def body(k, carry):
        off = k * 16
        rvv = rv_v[pl.ds(off, 16)]
        sg = idx_v[pl.ds(off, 16)]
        plsc.addupdate_scatter(sums_v, [sg], rvv)
        plsc.addupdate_scatter(cnts_v, [sg], one16)
        return carry

    lax.fori_loop(0, CH, body, 0)

    pltpu.sync_copy(sums_v.at[pl.ds(0, G)], sums_out.at[wid])
    pltpu.sync_copy(cnts_v.at[pl.ds(0, G)], cnts_out.at[wid])


def _sc_seg_call():
    return pl.kernel(
        _sc_seg_body,
        out_type=(jax.ShapeDtypeStruct((NW, G), jnp.float32),
                  jax.ShapeDtypeStruct((NW, G), jnp.float32)),
        mesh=plsc.VectorSubcoreMesh(core_axis_name="c", subcore_axis_name="s"),
        compiler_params=pltpu.CompilerParams(needs_layout_passes=False),
        scratch_types=[
            pltpu.VMEM((P,), jnp.float32),
            pltpu.VMEM((P,), jnp.int32),
            pltpu.VMEM((BINS,), jnp.float32),
            pltpu.VMEM((BINS,), jnp.float32),
        ],
    )


def _epilogue_body(s_ref, c_ref, bt_ref, wt_ref, b_ref, o_ref):
    c = jnp.sum(bt_ref[...] * wt_ref[...])
    s = jnp.sum(s_ref[...], axis=0, keepdims=True)
    cc = jnp.sum(c_ref[...], axis=0, keepdims=True)
    mean = s / jnp.maximum(cc, 1.0)
    o_ref[...] = mean + jnp.where(cc > 0.0, c, 0.0) + b_ref[...]


_epilogue_call = pl.pallas_call(
    _epilogue_body,
    out_shape=jax.ShapeDtypeStruct((1, G), jnp.float32),
)


@jax.jit
def kernel(x, batch_idx, y, ln_gamma, ln_beta, W, b):
    g2 = ln_gamma.reshape(1, D)
    bt2 = ln_beta.reshape(1, D)
    wt2 = W.reshape(1, D)

    rv = _rowval_call(x, g2, wt2)
    idx_p = jnp.pad(batch_idx.astype(jnp.int32), (0, NPAD - N),
                    constant_values=G)

    sums, cnts = _sc_seg_call()(rv.reshape(-1), idx_p)

    bb = jnp.broadcast_to(b.reshape(1, 1), (1, G))
    pred2 = _epilogue_call(sums, cnts, bt2, wt2, bb)
    return (pred2.reshape(G, 1), y)
